# baseline (speedup 1.0000x reference)
import functools

import jax
import jax.numpy as jnp
from jax import lax
from jax.experimental import pallas as pl
from jax.experimental.pallas import tpu as pltpu
from jax.experimental.pallas import tpu_sc as plsc

G = 2048
P = 1024
NB = 2
H = 64
E1 = 65536
E2 = 16384
NH = 2
HD = 32
FF = 512

NC = 2
NS = 16
NW = NC * NS
CPW_G = E1 // NW
CPW_P = E2 // NW
RG = G // NS
RP = P // NS
CHUNK = 128
HP = 128


def _lane_bcast(v16, j):
    return v16.at[jnp.full((16,), j, jnp.int32)].get(mode="promise_in_bounds")


DPT_G = E1 // NS
DPT_P = E2 // NS


def _newton_rsqrt(x):
    y = plsc.bitcast(jnp.int32(0x5F3759DF) - (plsc.bitcast(x, jnp.int32) >> 1),
                     jnp.float32)
    for _ in range(3):
        y = y * (1.5 - 0.5 * x * y * y)
    return y


def _aggdeg_body(sG_h, dG_h, wG_h, sP_h, dP_h, wP_h, xG_h, xP_h,
                 outG, outP, outDG, outDP,
                 sGv, dG2, wGv, sPv, dP2, wPv,
                 ddG, dwG, ddP, dwP, dinvG_v, dinvP_v, degf,
                 rows, rows2, stage, sem, sem2,
                 aggG_sh, aggP_sh, degG_sh, degP_sh):
    cid = lax.axis_index("c")
    sid = lax.axis_index("s")
    wid = cid * NS + sid

    def _zrow(r, _):
        for c in range(HP // 16):
            rows[r, pl.ds(c * 16, 16)] = jnp.zeros((16,), jnp.float32)
        return _
    lax.fori_loop(0, CHUNK, _zrow, None)
    pltpu.sync_copy(rows, aggG_sh.at[pl.ds(sid * RG, RG)])
    pltpu.sync_copy(rows.at[pl.ds(0, RP)], aggP_sh.at[pl.ds(sid * RP, RP)])
    pltpu.sync_copy(rows.at[0], degG_sh.at[pl.ds(sid * RG, RG)])
    pltpu.sync_copy(rows.at[0, pl.ds(0, RP)], degP_sh.at[pl.ds(sid * RP, RP)])

    base = wid * CPW_G
    pltpu.sync_copy(sG_h.at[pl.ds(base, CPW_G)], sGv)
    pltpu.sync_copy(wG_h.at[pl.ds(base, CPW_G)], wGv)
    for j in range(CPW_G // CHUNK):
        pltpu.sync_copy(dG_h.at[pl.ds(base + j * CHUNK, CHUNK)], dG2.at[j])
    basep = wid * CPW_P
    pltpu.sync_copy(sP_h.at[pl.ds(basep, CPW_P)], sPv)
    pltpu.sync_copy(wP_h.at[pl.ds(basep, CPW_P)], wPv)
    for j in range(CPW_P // CHUNK):
        pltpu.sync_copy(dP_h.at[pl.ds(basep + j * CHUNK, CHUNK)], dP2.at[j])

    dbase = sid * DPT_G
    for j in range(DPT_G // CHUNK):
        pltpu.sync_copy(dG_h.at[pl.ds(dbase + j * CHUNK, CHUNK)], ddG.at[j])
    pltpu.sync_copy(wG_h.at[pl.ds(dbase, DPT_G)], dwG)
    dbasep = sid * DPT_P
    for j in range(DPT_P // CHUNK):
        pltpu.sync_copy(dP_h.at[pl.ds(dbasep + j * CHUNK, CHUNK)], ddP.at[j])
    pltpu.sync_copy(wP_h.at[pl.ds(dbasep, DPT_P)], dwP)

    plsc.subcore_barrier()

    for j in range(DPT_G // CHUNK):
        pltpu.sync_copy(dwG.at[pl.ds(j * CHUNK, CHUNK)],
                        degG_sh.at[ddG.at[j]], add=True)
    for j in range(DPT_P // CHUNK):
        pltpu.sync_copy(dwP.at[pl.ds(j * CHUNK, CHUNK)],
                        degP_sh.at[ddP.at[j]], add=True)

    plsc.subcore_barrier()

    pltpu.sync_copy(degG_sh, degf.at[pl.ds(0, G)])
    def _dinvg(g, _):
        x = degf[pl.ds(g * 16, 16)] + 1.0
        dinvG_v[pl.ds(g * 16, 16)] = _newton_rsqrt(x)
        return _
    lax.fori_loop(0, G // 16, _dinvg, None)
    pltpu.sync_copy(degP_sh, degf.at[pl.ds(0, P)])
    def _dinvp(g, _):
        x = degf[pl.ds(g * 16, 16)] + 1.0
        dinvP_v[pl.ds(g * 16, 16)] = _newton_rsqrt(x)
        return _
    lax.fori_loop(0, P // 16, _dinvp, None)

    @pl.when(cid == 0)
    def _():
        pltpu.sync_copy(dinvG_v.at[pl.ds(sid * RG, RG)],
                        outDG.at[pl.ds(sid * RG, RG)])
        pltpu.sync_copy(dinvP_v.at[pl.ds(sid * RP, RP)],
                        outDP.at[pl.ds(sid * RP, RP)])

    def _pipe(y_h, sv, wv, d2, dinv_v, agg_sh, nchunks):
        def _start(j, buf, s):
            pltpu.make_async_copy(
                y_h.at[sv.at[pl.ds(j * CHUNK, CHUNK)]], buf, s).start()

        def _finish(j, buf, s):
            pltpu.make_async_copy(
                y_h.at[sv.at[pl.ds(j * CHUNK, CHUNK)]], buf, s).wait()
            for g in range(CHUNK // 16):
                s16 = sv[pl.ds(j * CHUNK + g * 16, 16)]
                w16 = wv[pl.ds(j * CHUNK + g * 16, 16)]
                w16 = w16 * plsc.load_gather(dinv_v, [s16])
                for jj in range(16):
                    bc = _lane_bcast(w16, jj)
                    r = g * 16 + jj
                    for c in range(H // 16):
                        buf[r, pl.ds(c * 16, 16)] = buf[r, pl.ds(c * 16, 16)] * bc
            pltpu.sync_copy(buf, agg_sh.at[d2.at[j]], add=True)

        _start(0, rows, sem)

        def _body(i, _):
            j = i * 2
            _start(j + 1, rows2, sem2)
            _finish(j, rows, sem)

            @pl.when(j + 2 < nchunks)
            def _():
                _start(j + 2, rows, sem)
            _finish(j + 1, rows2, sem2)
            return _

        lax.fori_loop(0, nchunks // 2, _body, None)

    _pipe(xG_h, sGv, wGv, dG2, dinvG_v, aggG_sh, CPW_G // CHUNK)
    _pipe(xP_h, sPv, wPv, dP2, dinvP_v, aggP_sh, CPW_P // CHUNK)

    plsc.subcore_barrier()

    pltpu.sync_copy(aggG_sh.at[pl.ds(sid * RG, RG)], stage)
    pltpu.sync_copy(stage, outG.at[cid, pl.ds(sid * RG, RG)])
    pltpu.sync_copy(aggP_sh.at[pl.ds(sid * RP, RP)], stage.at[pl.ds(0, RP)])
    pltpu.sync_copy(stage.at[pl.ds(0, RP)], outP.at[cid, pl.ds(sid * RP, RP)])


@functools.lru_cache(maxsize=None)
def _build_sc_aggdeg():
    return pl.kernel(
        _aggdeg_body,
        out_type=(jax.ShapeDtypeStruct((NC, G, HP), jnp.float32),
                  jax.ShapeDtypeStruct((NC, P, HP), jnp.float32),
                  jax.ShapeDtypeStruct((G,), jnp.float32),
                  jax.ShapeDtypeStruct((P,), jnp.float32)),
        mesh=plsc.VectorSubcoreMesh(core_axis_name="c", subcore_axis_name="s",
                                    num_cores=NC, num_subcores=NS),
        compiler_params=pltpu.CompilerParams(needs_layout_passes=False),
        scratch_types=[
            pltpu.VMEM((CPW_G,), jnp.int32),
            pltpu.VMEM((CPW_G // CHUNK, CHUNK), jnp.int32),
            pltpu.VMEM((CPW_G,), jnp.float32),
            pltpu.VMEM((CPW_P,), jnp.int32),
            pltpu.VMEM((CPW_P // CHUNK, CHUNK), jnp.int32),
            pltpu.VMEM((CPW_P,), jnp.float32),
            pltpu.VMEM((DPT_G // CHUNK, CHUNK), jnp.int32),
            pltpu.VMEM((DPT_G,), jnp.float32),
            pltpu.VMEM((DPT_P // CHUNK, CHUNK), jnp.int32),
            pltpu.VMEM((DPT_P,), jnp.float32),
            pltpu.VMEM((G,), jnp.float32),
            pltpu.VMEM((P,), jnp.float32),
            pltpu.VMEM((G,), jnp.float32),
            pltpu.VMEM((CHUNK, HP), jnp.float32),
            pltpu.VMEM((CHUNK, HP), jnp.float32),
            pltpu.VMEM((CHUNK, HP), jnp.float32),
            pltpu.SemaphoreType.DMA,
            pltpu.SemaphoreType.DMA,
            pltpu.VMEM_SHARED((G, HP), jnp.float32),
            pltpu.VMEM_SHARED((P, HP), jnp.float32),
            pltpu.VMEM_SHARED((G,), jnp.float32),
            pltpu.VMEM_SHARED((P,), jnp.float32),
        ],
    )


def _sc_aggdeg(*args):
    return _build_sc_aggdeg()(*args)


def _matT(x, w):
    return lax.dot_general(x, w, (((1,), (1,)), ((), ())),
                           preferred_element_type=jnp.float32)


def _bn(x, g, b):
    m = jnp.mean(x, axis=0, keepdims=True)
    v = jnp.mean((x - m) ** 2, axis=0, keepdims=True)
    return (x - m) / jnp.sqrt(v + 1e-5) * g + b


def _ln(x, g, b):
    m = jnp.mean(x, axis=-1, keepdims=True)
    v = jnp.mean((x - m) ** 2, axis=-1, keepdims=True)
    return (x - m) / jnp.sqrt(v + 1e-5) * g + b


def _mlp2(x, W1, b1, g1, be1, W2, b2, g2, be2):
    h = jnp.maximum(_bn(_matT(x, W1) + b1, g1, be1), 0.0)
    return _bn(_matT(h, W2) + b2, g2, be2)


def _maxnorm(x):
    n = jnp.sqrt(jnp.sum(x * x, axis=1, keepdims=True))
    return x * jnp.minimum(1.0, 1.0 / jnp.maximum(n, 1e-7))


def _pre_body(ge, ep, pe, mg_o, mp_o, mq_o, xg_o, xp_o):
    mg = _maxnorm(ge[...])
    mg_o[...] = mg
    mp_ = _maxnorm(ep[...])
    mp_o[...] = mp_
    mq = _maxnorm(pe[...])
    mq_o[...] = mq
    zg = jnp.zeros((G, HP - H), jnp.float32)
    xg_o[...] = jnp.concatenate([mp_, zg], axis=1)
    zp = jnp.zeros((P, HP - H), jnp.float32)
    xp_o[...] = jnp.concatenate([mq, zp], axis=1)


def _tc_pre(ge, ep, pe):
    return pl.pallas_call(
        _pre_body,
        out_shape=(
            jax.ShapeDtypeStruct((G, H), jnp.float32),
            jax.ShapeDtypeStruct((G, H), jnp.float32),
            jax.ShapeDtypeStruct((P, H), jnp.float32),
            jax.ShapeDtypeStruct((G, HP), jnp.float32),
            jax.ShapeDtypeStruct((P, HP), jnp.float32),
        ),
    )(ge, ep, pe)


def _main1_body(mg, mp_, mq, dinvG, dinvP, aggGp, aggPp, pidx,
                sgW, sgb, sgoW, sgob, bng, bnb,
                eW1, eb1, eg1, ebe1, eW2, eb2, eg2, ebe2,
                pW1, pb1, pg1, pbe1, pW2, pb2, pg2, pbe2,
                pbg, pbb,
                rW1, rb1, rg1, rbe1, rW2, rb2, rg2, rbe2,
                out_o):
    dG = dinvG[...]
    aggG = (aggGp[0, :, :H] + aggGp[1, :, :H]) * dG + mp_[...] * dG * dG
    posc1 = _matT(aggG, sgW[...]) + sgb[...]
    posc2 = _matT(mp_[...], sgW[...]) + sgb[...]

    mgv = mg[...]
    base0 = jnp.maximum(_bn(mgv, bng[...], bnb[...]), 0.0)
    bf = jnp.concatenate([base0 + 0.2 * posc1, base0 + 0.2 * posc2], axis=0)
    bf = _mlp2(bf, eW1[...], eb1[...], eg1[...], ebe1[...],
               eW2[...], eb2[...], eg2[...], ebe2[...])

    dP = dinvP[...]
    aggP = (aggPp[0, :, :H] + aggPp[1, :, :H]) * dP + mq[...] * dP * dP
    pgc = _matT(aggP, sgoW[...]) + sgob[...]

    iota = lax.broadcasted_iota(jnp.int32, (1, P), 1)
    oh_rows = []
    for b in range(NB):
        ohb = ((iota == pidx[b, 0]).astype(jnp.float32)
               + (iota == pidx[b, 1]).astype(jnp.float32))
        oh_rows.append(ohb)
    oh = jnp.concatenate(oh_rows, axis=0)
    psum = jnp.dot(oh, pgc, preferred_element_type=jnp.float32)
    etot = _mlp2(psum, pW1[...], pb1[...], pg1[...], pbe1[...],
                 pW2[...], pb2[...], pg2[...], pbe2[...])

    efull = jnp.concatenate(
        [jnp.broadcast_to(etot[b:b + 1, :], (G, H)) for b in range(NB)],
        axis=0)
    base = bf + efull
    base = jnp.maximum(_bn(base, pbg[...], pbb[...]), 0.0)
    out_o[...] = _mlp2(base, rW1[...], rb1[...], rg1[...], rbe1[...],
                       rW2[...], rb2[...], rg2[...], rbe2[...])


def _tc_main1(mg, mp_, mq, dinvG, dinvP, aggGp, aggPp, pidx, wlist):
    in_specs = ([pl.BlockSpec()] * 7
                + [pl.BlockSpec(memory_space=pltpu.SMEM)]
                + [pl.BlockSpec()] * len(wlist))
    return pl.pallas_call(
        _main1_body,
        out_shape=jax.ShapeDtypeStruct((NB * G, H), jnp.float32),
        in_specs=in_specs,
    )(mg, mp_, mq, dinvG, dinvP, aggGp, aggPp, pidx, *wlist)


QCH = 512


def _main2_body(out1, mask, xcol, epsT,
                wq, bq, wk, bk, wv, bv, wo, bo,
                ln1g, ln1b, ffW1, ffb1, ffW2, ffb2, ln2g, ln2b,
                indvW, indvB,
                encW, encb, muW, mub, lvW, lvb, dW1, db1, dW2, db2,
                outf, klo):
    b = pl.program_id(0)
    ob = out1[...]
    q = _matT(ob, wq[...]) + bq[...]
    k = _matT(ob, wk[...]) + bk[...]
    v = _matT(ob, wv[...]) + bv[...]
    scale = 1.0 / jnp.sqrt(jnp.float32(HD))
    heads = []
    for h in range(NH):
        qh = q[:, h * HD:(h + 1) * HD]
        kh = k[:, h * HD:(h + 1) * HD]
        vh = v[:, h * HD:(h + 1) * HD]
        oh_chunks = []
        for qb in range(G // QCH):
            att = lax.dot_general(qh[qb * QCH:(qb + 1) * QCH, :], kh,
                                  (((1,), (1,)), ((), ())),
                                  preferred_element_type=jnp.float32)
            att = att * scale + mask[pl.ds(qb * QCH, QCH), :]
            att = att - jnp.max(att, axis=-1, keepdims=True)
            att = jnp.exp(att)
            att = att / jnp.sum(att, axis=-1, keepdims=True)
            oh_chunks.append(jnp.dot(att, vh,
                                     preferred_element_type=jnp.float32))
        heads.append(jnp.concatenate(oh_chunks, axis=0))
    o = jnp.concatenate(heads, axis=1)
    o = _matT(o, wo[...]) + bo[...]
    h1 = _ln(ob + o, ln1g[...], ln1b[...])
    ff = _matT(jnp.maximum(_matT(h1, ffW1[...]) + ffb1[...], 0.0),
               ffW2[...]) + ffb2[...]
    outp = _ln(h1 + ff, ln2g[...], ln2b[...])
    wsum = jnp.sum(outp * indvW[...], axis=1, keepdims=True) + indvB[...]

    x2 = xcol[...]
    h1v = jnp.maximum(jnp.dot(encW[...], x2,
                              preferred_element_type=jnp.float32)
                      + encb[...], 0.0)
    mu = jnp.dot(muW[...], h1v, preferred_element_type=jnp.float32) + mub[...]
    lv = jnp.dot(lvW[...], h1v, preferred_element_type=jnp.float32) + lvb[...]
    z = mu + epsT[0] * jnp.exp(0.5 * lv)
    d1 = jnp.maximum(jnp.dot(dW1[...], z,
                             preferred_element_type=jnp.float32)
                     + db1[...], 0.0)
    recon = jnp.dot(dW2[...], d1,
                    preferred_element_type=jnp.float32) + db2[...]
    recon = jnp.where(x2 == 0.0, 0.0, recon)
    outf[...] = (wsum + recon)[None]

    klb = -0.5 * jnp.sum(1.0 + lv - mu * mu - jnp.exp(lv))

    @pl.when(b == 0)
    def _():
        klo[...] = jnp.zeros((1, 1), jnp.float32)
    klo[...] += jnp.reshape(klb / NB, (1, 1))


def _tc_main2(out1, mask, xcol, epsT, wlist):
    full = lambda s: pl.BlockSpec(s, lambda b: (0,) * len(s))
    in_specs = [
        pl.BlockSpec((G, H), lambda b: (b, 0)),
        full((G, G)),
        pl.BlockSpec((G, 1), lambda b: (b, 0)),
        pl.BlockSpec((1, 16, 1), lambda b: (b, 0, 0)),
    ] + [full(w.shape) for w in wlist]
    return pl.pallas_call(
        _main2_body,
        grid=(NB,),
        out_shape=(jax.ShapeDtypeStruct((NB, G, 1), jnp.float32),
                   jax.ShapeDtypeStruct((1, 1), jnp.float32)),
        in_specs=in_specs,
        out_specs=(pl.BlockSpec((1, G, 1), lambda b: (b, 0, 0)),
                   pl.BlockSpec((1, 1), lambda b: (0, 0))),
    )(out1, mask, xcol, epsT, *wlist)


def kernel(x, pert_idx, batch, mask, G_coexpress, G_coexpress_weight,
           G_go, G_go_weight, params):
    p = params
    row = lambda a: a.reshape(1, -1)
    col = lambda a: a.reshape(-1, 1)
    i32 = jnp.int32

    sG = G_coexpress[0].astype(i32)
    dG = G_coexpress[1].astype(i32)
    wG = G_coexpress_weight.astype(jnp.float32)
    sP = G_go[0].astype(i32)
    dP = G_go[1].astype(i32)
    wP = G_go_weight.astype(jnp.float32)

    mg, mp_, mq, xG, xP = _tc_pre(p['gene_emb'], p['emb_pos'], p['pert_emb'])

    aggGp, aggPp, dinvGf, dinvPf = _sc_aggdeg(sG, dG, wG, sP, dP, wP, xG, xP)
    dinvG = dinvGf[:, None]
    dinvP = dinvPf[:, None]

    w1 = [p['sg_gene_W'], row(p['sg_gene_b']), p['sg_go_W'], row(p['sg_go_b']),
          row(p['bn_emb_g']), row(p['bn_emb_b']),
          p['etv2_W1'], row(p['etv2_b1']), row(p['etv2_g1']), row(p['etv2_be1']),
          p['etv2_W2'], row(p['etv2_b2']), row(p['etv2_g2']), row(p['etv2_be2']),
          p['pf_W1'], row(p['pf_b1']), row(p['pf_g1']), row(p['pf_be1']),
          p['pf_W2'], row(p['pf_b2']), row(p['pf_g2']), row(p['pf_be2']),
          row(p['bn_pb_g']), row(p['bn_pb_b']),
          p['rec_W1'], row(p['rec_b1']), row(p['rec_g1']), row(p['rec_be1']),
          p['rec_W2'], row(p['rec_b2']), row(p['rec_g2']), row(p['rec_be2'])]
    out1 = _tc_main1(mg, mp_, mq, dinvG, dinvP, aggGp, aggPp,
                     pert_idx.astype(i32), w1)

    w2 = [p['te_Wq'], row(p['te_bq']), p['te_Wk'], row(p['te_bk']),
          p['te_Wv'], row(p['te_bv']), p['te_Wo'], row(p['te_bo']),
          row(p['te_ln1_g']), row(p['te_ln1_b']),
          p['te_ff_W1'], row(p['te_ff_b1']), p['te_ff_W2'], row(p['te_ff_b2']),
          row(p['te_ln2_g']), row(p['te_ln2_b']),
          p['indv_w1'].reshape(G, H), col(p['indv_b1']),
          p['vae_enc_W'], col(p['vae_enc_b']),
          p['vae_mu_W'], col(p['vae_mu_b']),
          p['vae_lv_W'], col(p['vae_lv_b']),
          p['vae_dec_W1'], col(p['vae_dec_b1']),
          p['vae_dec_W2'], col(p['vae_dec_b2'])]
    outf, kl = _tc_main2(out1, mask, x.reshape(NB * G, 1),
                         p['vae_eps'][:, :, None], w2)

    return (outf.reshape(NB, G), kl.reshape(()))


# speedup vs baseline: 1.2483x; 1.2483x over previous
import functools

import jax
import jax.numpy as jnp
from jax import lax
from jax.experimental import pallas as pl
from jax.experimental.pallas import tpu as pltpu
from jax.experimental.pallas import tpu_sc as plsc

G = 2048
P = 1024
NB = 2
H = 64
E1 = 65536
E2 = 16384
NH = 2
HD = 32
FF = 512

NC = 2
NS = 16
NW = NC * NS
CPW_G = E1 // NW
CPW_P = E2 // NW
RG = G // NS
RP = P // NS
CHUNK = 128
HP = 128


def _lane_bcast(v16, j):
    return v16.at[jnp.full((16,), j, jnp.int32)].get(mode="promise_in_bounds")


DPT_G = E1 // NS
DPT_P = E2 // NS


def _newton_rsqrt(x):
    y = plsc.bitcast(jnp.int32(0x5F3759DF) - (plsc.bitcast(x, jnp.int32) >> 1),
                     jnp.float32)
    for _ in range(3):
        y = y * (1.5 - 0.5 * x * y * y)
    return y


def _aggdeg_body(sG_h, dG_h, wG_h, sP_h, dP_h, wP_h, xG_h, xP_h,
                 outG, outP, outDG, outDP,
                 sGv, dG2, wGv, sPv, dP2, wPv,
                 ddG, dwG, ddP, dwP, dinvG_v, dinvP_v, degf,
                 rows, rows2, stage, sem, sem2,
                 aggG_sh, aggP_sh, degG_sh, degP_sh):
    cid = lax.axis_index("c")
    sid = lax.axis_index("s")
    wid = cid * NS + sid

    stg = []

    def _fire(src, dst):
        pltpu.make_async_copy(src, dst, sem2).start()
        stg.append((src, dst))

    def _drain():
        for src, dst in stg:
            pltpu.make_async_copy(src, dst, sem2).wait()
        stg.clear()

    dbase = sid * DPT_G
    for j in range(DPT_G // CHUNK):
        _fire(dG_h.at[pl.ds(dbase + j * CHUNK, CHUNK)], ddG.at[j])
    _fire(wG_h.at[pl.ds(dbase, DPT_G)], dwG)
    dbasep = sid * DPT_P
    for j in range(DPT_P // CHUNK):
        _fire(dP_h.at[pl.ds(dbasep + j * CHUNK, CHUNK)], ddP.at[j])
    _fire(wP_h.at[pl.ds(dbasep, DPT_P)], dwP)

    base = wid * CPW_G
    _fire(sG_h.at[pl.ds(base, CPW_G)], sGv)
    _fire(wG_h.at[pl.ds(base, CPW_G)], wGv)
    for j in range(CPW_G // CHUNK):
        _fire(dG_h.at[pl.ds(base + j * CHUNK, CHUNK)], dG2.at[j])
    basep = wid * CPW_P
    _fire(sP_h.at[pl.ds(basep, CPW_P)], sPv)
    _fire(wP_h.at[pl.ds(basep, CPW_P)], wPv)
    for j in range(CPW_P // CHUNK):
        _fire(dP_h.at[pl.ds(basep + j * CHUNK, CHUNK)], dP2.at[j])

    def _zrow(r, _):
        for c in range(HP // 16):
            rows[r, pl.ds(c * 16, 16)] = jnp.zeros((16,), jnp.float32)
        return _
    lax.fori_loop(0, CHUNK, _zrow, None)
    pltpu.sync_copy(rows, aggG_sh.at[pl.ds(sid * RG, RG)])
    pltpu.sync_copy(rows.at[pl.ds(0, RP)], aggP_sh.at[pl.ds(sid * RP, RP)])
    pltpu.sync_copy(rows.at[0], degG_sh.at[pl.ds(sid * RG, RG)])
    pltpu.sync_copy(rows.at[0, pl.ds(0, RP)], degP_sh.at[pl.ds(sid * RP, RP)])

    _drain()
    plsc.subcore_barrier()

    deg_streams = []
    for j in range(DPT_G // CHUNK):
        deg_streams.append((dwG.at[pl.ds(j * CHUNK, CHUNK)],
                            degG_sh.at[ddG.at[j]]))
    for j in range(DPT_P // CHUNK):
        deg_streams.append((dwP.at[pl.ds(j * CHUNK, CHUNK)],
                            degP_sh.at[ddP.at[j]]))
    for src, dst in deg_streams:
        pltpu.make_async_copy(src, dst, sem2).start(add=True)
    for src, dst in deg_streams:
        pltpu.make_async_copy(src, dst, sem2).wait()

    plsc.subcore_barrier()

    pltpu.sync_copy(degG_sh, degf.at[pl.ds(0, G)])
    def _dinvg(g, _):
        x = degf[pl.ds(g * 16, 16)] + 1.0
        dinvG_v[pl.ds(g * 16, 16)] = _newton_rsqrt(x)
        return _
    lax.fori_loop(0, G // 16, _dinvg, None)
    pltpu.sync_copy(degP_sh, degf.at[pl.ds(0, P)])
    def _dinvp(g, _):
        x = degf[pl.ds(g * 16, 16)] + 1.0
        dinvP_v[pl.ds(g * 16, 16)] = _newton_rsqrt(x)
        return _
    lax.fori_loop(0, P // 16, _dinvp, None)

    @pl.when(cid == 0)
    def _():
        pltpu.sync_copy(dinvG_v.at[pl.ds(sid * RG, RG)],
                        outDG.at[pl.ds(sid * RG, RG)])
        pltpu.sync_copy(dinvP_v.at[pl.ds(sid * RP, RP)],
                        outDP.at[pl.ds(sid * RP, RP)])

    def _pipe(y_h, sv, wv, d2, dinv_v, agg_sh, nchunks):
        def _start(j, buf, s):
            pltpu.make_async_copy(
                y_h.at[sv.at[pl.ds(j * CHUNK, CHUNK)]], buf, s).start()

        def _finish(j, buf, s):
            pltpu.make_async_copy(
                y_h.at[sv.at[pl.ds(j * CHUNK, CHUNK)]], buf, s).wait()
            for g in range(CHUNK // 16):
                s16 = sv[pl.ds(j * CHUNK + g * 16, 16)]
                w16 = wv[pl.ds(j * CHUNK + g * 16, 16)]
                w16 = w16 * plsc.load_gather(dinv_v, [s16])
                for jj in range(16):
                    bc = _lane_bcast(w16, jj)
                    r = g * 16 + jj
                    for c in range(H // 16):
                        buf[r, pl.ds(c * 16, 16)] = buf[r, pl.ds(c * 16, 16)] * bc
            pltpu.sync_copy(buf, agg_sh.at[d2.at[j]], add=True)

        _start(0, rows, sem)

        def _body(i, _):
            j = i * 2
            _start(j + 1, rows2, sem2)
            _finish(j, rows, sem)

            @pl.when(j + 2 < nchunks)
            def _():
                _start(j + 2, rows, sem)
            _finish(j + 1, rows2, sem2)
            return _

        lax.fori_loop(0, nchunks // 2, _body, None)

    _pipe(xG_h, sGv, wGv, dG2, dinvG_v, aggG_sh, CPW_G // CHUNK)
    _pipe(xP_h, sPv, wPv, dP2, dinvP_v, aggP_sh, CPW_P // CHUNK)

    plsc.subcore_barrier()

    pltpu.sync_copy(aggG_sh.at[pl.ds(sid * RG, RG)], stage)
    pltpu.sync_copy(stage, outG.at[cid, pl.ds(sid * RG, RG)])
    pltpu.sync_copy(aggP_sh.at[pl.ds(sid * RP, RP)], stage.at[pl.ds(0, RP)])
    pltpu.sync_copy(stage.at[pl.ds(0, RP)], outP.at[cid, pl.ds(sid * RP, RP)])


@functools.lru_cache(maxsize=None)
def _build_sc_aggdeg():
    return pl.kernel(
        _aggdeg_body,
        out_type=(jax.ShapeDtypeStruct((NC, G, HP), jnp.float32),
                  jax.ShapeDtypeStruct((NC, P, HP), jnp.float32),
                  jax.ShapeDtypeStruct((G,), jnp.float32),
                  jax.ShapeDtypeStruct((P,), jnp.float32)),
        mesh=plsc.VectorSubcoreMesh(core_axis_name="c", subcore_axis_name="s",
                                    num_cores=NC, num_subcores=NS),
        compiler_params=pltpu.CompilerParams(needs_layout_passes=False),
        scratch_types=[
            pltpu.VMEM((CPW_G,), jnp.int32),
            pltpu.VMEM((CPW_G // CHUNK, CHUNK), jnp.int32),
            pltpu.VMEM((CPW_G,), jnp.float32),
            pltpu.VMEM((CPW_P,), jnp.int32),
            pltpu.VMEM((CPW_P // CHUNK, CHUNK), jnp.int32),
            pltpu.VMEM((CPW_P,), jnp.float32),
            pltpu.VMEM((DPT_G // CHUNK, CHUNK), jnp.int32),
            pltpu.VMEM((DPT_G,), jnp.float32),
            pltpu.VMEM((DPT_P // CHUNK, CHUNK), jnp.int32),
            pltpu.VMEM((DPT_P,), jnp.float32),
            pltpu.VMEM((G,), jnp.float32),
            pltpu.VMEM((P,), jnp.float32),
            pltpu.VMEM((G,), jnp.float32),
            pltpu.VMEM((CHUNK, HP), jnp.float32),
            pltpu.VMEM((CHUNK, HP), jnp.float32),
            pltpu.VMEM((CHUNK, HP), jnp.float32),
            pltpu.SemaphoreType.DMA,
            pltpu.SemaphoreType.DMA,
            pltpu.VMEM_SHARED((G, HP), jnp.float32),
            pltpu.VMEM_SHARED((P, HP), jnp.float32),
            pltpu.VMEM_SHARED((G,), jnp.float32),
            pltpu.VMEM_SHARED((P,), jnp.float32),
        ],
    )


def _sc_aggdeg(*args):
    return _build_sc_aggdeg()(*args)


def _matT(x, w):
    return lax.dot_general(x, w, (((1,), (1,)), ((), ())),
                           preferred_element_type=jnp.float32)


def _bn(x, g, b):
    m = jnp.mean(x, axis=0, keepdims=True)
    v = jnp.mean((x - m) ** 2, axis=0, keepdims=True)
    return (x - m) / jnp.sqrt(v + 1e-5) * g + b


def _ln(x, g, b):
    m = jnp.mean(x, axis=-1, keepdims=True)
    v = jnp.mean((x - m) ** 2, axis=-1, keepdims=True)
    return (x - m) / jnp.sqrt(v + 1e-5) * g + b


def _mlp2(x, W1, b1, g1, be1, W2, b2, g2, be2):
    h = jnp.maximum(_bn(_matT(x, W1) + b1, g1, be1), 0.0)
    return _bn(_matT(h, W2) + b2, g2, be2)


def _maxnorm(x):
    n = jnp.sqrt(jnp.sum(x * x, axis=1, keepdims=True))
    return x * jnp.minimum(1.0, 1.0 / jnp.maximum(n, 1e-7))


def _pre_body(ge, ep, pe, mg_o, mp_o, mq_o, xg_o, xp_o):
    mg = _maxnorm(ge[...])
    mg_o[...] = mg
    mp_ = _maxnorm(ep[...])
    mp_o[...] = mp_
    mq = _maxnorm(pe[...])
    mq_o[...] = mq
    zg = jnp.zeros((G, HP - H), jnp.float32)
    xg_o[...] = jnp.concatenate([mp_, zg], axis=1)
    zp = jnp.zeros((P, HP - H), jnp.float32)
    xp_o[...] = jnp.concatenate([mq, zp], axis=1)


def _tc_pre(ge, ep, pe):
    return pl.pallas_call(
        _pre_body,
        out_shape=(
            jax.ShapeDtypeStruct((G, H), jnp.float32),
            jax.ShapeDtypeStruct((G, H), jnp.float32),
            jax.ShapeDtypeStruct((P, H), jnp.float32),
            jax.ShapeDtypeStruct((G, HP), jnp.float32),
            jax.ShapeDtypeStruct((P, HP), jnp.float32),
        ),
    )(ge, ep, pe)


def _main1_body(mg, mp_, mq, dinvG, dinvP, aggGp, aggPp, pidx,
                sgW, sgb, sgoW, sgob, bng, bnb,
                eW1, eb1, eg1, ebe1, eW2, eb2, eg2, ebe2,
                pW1, pb1, pg1, pbe1, pW2, pb2, pg2, pbe2,
                pbg, pbb,
                rW1, rb1, rg1, rbe1, rW2, rb2, rg2, rbe2,
                out_o):
    dG = dinvG[...]
    aggG = (aggGp[0, :, :H] + aggGp[1, :, :H]) * dG + mp_[...] * dG * dG
    posc1 = _matT(aggG, sgW[...]) + sgb[...]
    posc2 = _matT(mp_[...], sgW[...]) + sgb[...]

    mgv = mg[...]
    base0 = jnp.maximum(_bn(mgv, bng[...], bnb[...]), 0.0)
    bf = jnp.concatenate([base0 + 0.2 * posc1, base0 + 0.2 * posc2], axis=0)
    bf = _mlp2(bf, eW1[...], eb1[...], eg1[...], ebe1[...],
               eW2[...], eb2[...], eg2[...], ebe2[...])

    dP = dinvP[...]
    aggP = (aggPp[0, :, :H] + aggPp[1, :, :H]) * dP + mq[...] * dP * dP
    pgc = _matT(aggP, sgoW[...]) + sgob[...]

    iota = lax.broadcasted_iota(jnp.int32, (1, P), 1)
    oh_rows = []
    for b in range(NB):
        ohb = ((iota == pidx[b, 0]).astype(jnp.float32)
               + (iota == pidx[b, 1]).astype(jnp.float32))
        oh_rows.append(ohb)
    oh = jnp.concatenate(oh_rows, axis=0)
    psum = jnp.dot(oh, pgc, preferred_element_type=jnp.float32)
    etot = _mlp2(psum, pW1[...], pb1[...], pg1[...], pbe1[...],
                 pW2[...], pb2[...], pg2[...], pbe2[...])

    efull = jnp.concatenate(
        [jnp.broadcast_to(etot[b:b + 1, :], (G, H)) for b in range(NB)],
        axis=0)
    base = bf + efull
    base = jnp.maximum(_bn(base, pbg[...], pbb[...]), 0.0)
    out_o[...] = _mlp2(base, rW1[...], rb1[...], rg1[...], rbe1[...],
                       rW2[...], rb2[...], rg2[...], rbe2[...])


def _tc_main1(mg, mp_, mq, dinvG, dinvP, aggGp, aggPp, pidx, wlist):
    in_specs = ([pl.BlockSpec()] * 7
                + [pl.BlockSpec(memory_space=pltpu.SMEM)]
                + [pl.BlockSpec()] * len(wlist))
    return pl.pallas_call(
        _main1_body,
        out_shape=jax.ShapeDtypeStruct((NB * G, H), jnp.float32),
        in_specs=in_specs,
    )(mg, mp_, mq, dinvG, dinvP, aggGp, aggPp, pidx, *wlist)


QCH = 512


def _main2_body(out1, mask, xcol, epsT,
                wq, bq, wk, bk, wv, bv, wo, bo,
                ln1g, ln1b, ffW1, ffb1, ffW2, ffb2, ln2g, ln2b,
                indvW, indvB,
                encW, encb, muW, mub, lvW, lvb, dW1, db1, dW2, db2,
                outf, klo):
    b = pl.program_id(0)
    ob = out1[...]
    q = _matT(ob, wq[...]) + bq[...]
    k = _matT(ob, wk[...]) + bk[...]
    v = _matT(ob, wv[...]) + bv[...]
    scale = 1.0 / jnp.sqrt(jnp.float32(HD))
    heads = []
    for h in range(NH):
        qh = q[:, h * HD:(h + 1) * HD]
        kh = k[:, h * HD:(h + 1) * HD]
        vh = v[:, h * HD:(h + 1) * HD]
        oh_chunks = []
        for qb in range(G // QCH):
            att = lax.dot_general(qh[qb * QCH:(qb + 1) * QCH, :], kh,
                                  (((1,), (1,)), ((), ())),
                                  preferred_element_type=jnp.float32)
            att = att * scale + mask[pl.ds(qb * QCH, QCH), :]
            att = att - jnp.max(att, axis=-1, keepdims=True)
            att = jnp.exp(att)
            att = att / jnp.sum(att, axis=-1, keepdims=True)
            oh_chunks.append(jnp.dot(att, vh,
                                     preferred_element_type=jnp.float32))
        heads.append(jnp.concatenate(oh_chunks, axis=0))
    o = jnp.concatenate(heads, axis=1)
    o = _matT(o, wo[...]) + bo[...]
    h1 = _ln(ob + o, ln1g[...], ln1b[...])
    ff = _matT(jnp.maximum(_matT(h1, ffW1[...]) + ffb1[...], 0.0),
               ffW2[...]) + ffb2[...]
    outp = _ln(h1 + ff, ln2g[...], ln2b[...])
    wsum = jnp.sum(outp * indvW[...], axis=1, keepdims=True) + indvB[...]

    x2 = xcol[...]
    h1v = jnp.maximum(jnp.dot(encW[...], x2,
                              preferred_element_type=jnp.float32)
                      + encb[...], 0.0)
    mu = jnp.dot(muW[...], h1v, preferred_element_type=jnp.float32) + mub[...]
    lv = jnp.dot(lvW[...], h1v, preferred_element_type=jnp.float32) + lvb[...]
    z = mu + epsT[0] * jnp.exp(0.5 * lv)
    d1 = jnp.maximum(jnp.dot(dW1[...], z,
                             preferred_element_type=jnp.float32)
                     + db1[...], 0.0)
    recon = jnp.dot(dW2[...], d1,
                    preferred_element_type=jnp.float32) + db2[...]
    recon = jnp.where(x2 == 0.0, 0.0, recon)
    outf[...] = (wsum + recon)[None]

    klb = -0.5 * jnp.sum(1.0 + lv - mu * mu - jnp.exp(lv))

    @pl.when(b == 0)
    def _():
        klo[...] = jnp.zeros((1, 1), jnp.float32)
    klo[...] += jnp.reshape(klb / NB, (1, 1))


def _tc_main2(out1, mask, xcol, epsT, wlist):
    full = lambda s: pl.BlockSpec(s, lambda b: (0,) * len(s))
    in_specs = [
        pl.BlockSpec((G, H), lambda b: (b, 0)),
        full((G, G)),
        pl.BlockSpec((G, 1), lambda b: (b, 0)),
        pl.BlockSpec((1, 16, 1), lambda b: (b, 0, 0)),
    ] + [full(w.shape) for w in wlist]
    return pl.pallas_call(
        _main2_body,
        grid=(NB,),
        out_shape=(jax.ShapeDtypeStruct((NB, G, 1), jnp.float32),
                   jax.ShapeDtypeStruct((1, 1), jnp.float32)),
        in_specs=in_specs,
        out_specs=(pl.BlockSpec((1, G, 1), lambda b: (b, 0, 0)),
                   pl.BlockSpec((1, 1), lambda b: (0, 0))),
    )(out1, mask, xcol, epsT, *wlist)


def kernel(x, pert_idx, batch, mask, G_coexpress, G_coexpress_weight,
           G_go, G_go_weight, params):
    p = params
    row = lambda a: a.reshape(1, -1)
    col = lambda a: a.reshape(-1, 1)
    i32 = jnp.int32

    sG = G_coexpress[0].astype(i32)
    dG = G_coexpress[1].astype(i32)
    wG = G_coexpress_weight.astype(jnp.float32)
    sP = G_go[0].astype(i32)
    dP = G_go[1].astype(i32)
    wP = G_go_weight.astype(jnp.float32)

    mg, mp_, mq, xG, xP = _tc_pre(p['gene_emb'], p['emb_pos'], p['pert_emb'])

    aggGp, aggPp, dinvGf, dinvPf = _sc_aggdeg(sG, dG, wG, sP, dP, wP, xG, xP)
    dinvG = dinvGf[:, None]
    dinvP = dinvPf[:, None]

    w1 = [p['sg_gene_W'], row(p['sg_gene_b']), p['sg_go_W'], row(p['sg_go_b']),
          row(p['bn_emb_g']), row(p['bn_emb_b']),
          p['etv2_W1'], row(p['etv2_b1']), row(p['etv2_g1']), row(p['etv2_be1']),
          p['etv2_W2'], row(p['etv2_b2']), row(p['etv2_g2']), row(p['etv2_be2']),
          p['pf_W1'], row(p['pf_b1']), row(p['pf_g1']), row(p['pf_be1']),
          p['pf_W2'], row(p['pf_b2']), row(p['pf_g2']), row(p['pf_be2']),
          row(p['bn_pb_g']), row(p['bn_pb_b']),
          p['rec_W1'], row(p['rec_b1']), row(p['rec_g1']), row(p['rec_be1']),
          p['rec_W2'], row(p['rec_b2']), row(p['rec_g2']), row(p['rec_be2'])]
    out1 = _tc_main1(mg, mp_, mq, dinvG, dinvP, aggGp, aggPp,
                     pert_idx.astype(i32), w1)

    w2 = [p['te_Wq'], row(p['te_bq']), p['te_Wk'], row(p['te_bk']),
          p['te_Wv'], row(p['te_bv']), p['te_Wo'], row(p['te_bo']),
          row(p['te_ln1_g']), row(p['te_ln1_b']),
          p['te_ff_W1'], row(p['te_ff_b1']), p['te_ff_W2'], row(p['te_ff_b2']),
          row(p['te_ln2_g']), row(p['te_ln2_b']),
          p['indv_w1'].reshape(G, H), col(p['indv_b1']),
          p['vae_enc_W'], col(p['vae_enc_b']),
          p['vae_mu_W'], col(p['vae_mu_b']),
          p['vae_lv_W'], col(p['vae_lv_b']),
          p['vae_dec_W1'], col(p['vae_dec_b1']),
          p['vae_dec_W2'], col(p['vae_dec_b2'])]
    outf, kl = _tc_main2(out1, mask, x.reshape(NB * G, 1),
                         p['vae_eps'][:, :, None], w2)

    return (outf.reshape(NB, G), kl.reshape(()))
